# baseline (speedup 1.0000x reference)
import functools

import jax
import jax.numpy as jnp
from jax import lax
from jax.experimental import pallas as pl
from jax.experimental.pallas import tpu as pltpu
from jax.experimental.pallas import tpu_sc as plsc

_N = 10000
_NP = 10112
_NT = 16
_NC = 2
_NW = _NT * _NC
_RPT = _NP // _NT
_CH = 512
_NCH = 20
_NBUF = 2
_BN = 2048


def _make_sc_scatter(d: int):
  mesh = plsc.VectorSubcoreMesh(core_axis_name="c", subcore_axis_name="s")

  @functools.partial(
      pl.kernel,
      out_type=jax.ShapeDtypeStruct((_NC, _NP, d), jnp.float32),
      mesh=mesh,
      compiler_params=pltpu.CompilerParams(use_tc_tiling_on_sc=False),
      scratch_types=[
          pltpu.VMEM((_NCH, _CH), jnp.int32),
          pltpu.VMEM((_NCH, _CH), jnp.int32),
          pltpu.VMEM((_NBUF, _CH, d), jnp.float32),
          pltpu.VMEM_SHARED((_NP, d), jnp.float32),
          pltpu.VMEM_SHARED((_NP, d), jnp.float32),
          [pltpu.SemaphoreType.DMA] * _NBUF,
          [pltpu.SemaphoreType.DMA] * _NBUF,
      ],
  )
  def sc_scatter(table_hbm, src_hbm, dst_hbm, zeros_hbm, out_hbm,
                 src_v, dst_v, rows_v, acc, table_spm, gsems, ssems):
    cid = lax.axis_index("c")
    sid = lax.axis_index("s")
    wid = sid * _NC + cid
    r0 = sid * _RPT
    pltpu.sync_copy(zeros_hbm.at[pl.ds(r0, _RPT)], acc.at[pl.ds(r0, _RPT)])
    pltpu.sync_copy(table_hbm.at[pl.ds(r0, _RPT)], table_spm.at[pl.ds(r0, _RPT)])
    pltpu.sync_copy(src_hbm.at[wid], src_v)
    pltpu.sync_copy(dst_hbm.at[wid], dst_v)
    plsc.subcore_barrier()

    gh = [None] * _NBUF
    sh = [None] * _NBUF

    def issue_gather(c):
      b = c % _NBUF
      if sh[b] is not None:
        sh[b].wait()
        sh[b] = None
      gh[b] = pltpu.async_copy(
          table_spm.at[src_v.at[c]], rows_v.at[b], gsems[b])

    for c in range(min(_NBUF, _NCH)):
      issue_gather(c)
    for c in range(_NCH):
      b = c % _NBUF
      gh[b].wait()
      sh[b] = pltpu.async_copy(rows_v.at[b], acc.at[dst_v.at[c]], ssems[b],
                               add=True)
      if c + _NBUF < _NCH:
        issue_gather(c + _NBUF)
    for b in range(_NBUF):
      if sh[b] is not None:
        sh[b].wait()
    plsc.subcore_barrier()
    pltpu.sync_copy(acc.at[pl.ds(r0, _RPT)],
                    out_hbm.at[cid].at[pl.ds(r0, _RPT)])

  return sc_scatter


def _make_sc_degree():
  mesh = plsc.VectorSubcoreMesh(core_axis_name="c", subcore_axis_name="s")

  @functools.partial(
      pl.kernel,
      out_type=jax.ShapeDtypeStruct((_NC, _NP, 8), jnp.float32),
      mesh=mesh,
      compiler_params=pltpu.CompilerParams(use_tc_tiling_on_sc=False,
                                           needs_layout_passes=False),
      scratch_types=[
          pltpu.VMEM((_NCH, _CH), jnp.int32),
          pltpu.VMEM((_NP,), jnp.float32),
          pltpu.VMEM((_NT, 640), jnp.float32),
          pltpu.VMEM((640,), jnp.float32),
          pltpu.VMEM((640, 16), jnp.float32),
          pltpu.VMEM_SHARED((_NT, _NP), jnp.float32),
      ],
  )
  def sc_degree(dst_hbm, zeros_hbm, out_hbm, dst_v, degl, buf, accl, repl,
                stage):
    cid = lax.axis_index("c")
    sid = lax.axis_index("s")
    wid = sid * _NC + cid
    r0 = sid * _RPT
    pltpu.sync_copy(dst_hbm.at[wid], dst_v)
    pltpu.sync_copy(zeros_hbm, degl)
    ones = jnp.ones((16,), jnp.float32)

    def body(r, carry):
      for k in range(_CH // 16):
        idx = dst_v[r, pl.ds(k * 16, 16)]
        plsc.addupdate_scatter(degl, [idx], ones)
      return carry

    lax.fori_loop(0, _NCH, body, 0)
    pltpu.sync_copy(degl, stage.at[sid])
    plsc.subcore_barrier()
    for t in range(_NT):
      pltpu.sync_copy(stage.at[t].at[pl.ds(r0, _RPT)],
                      buf.at[t].at[pl.ds(0, _RPT)])
    for c in range(640 // 16):
      v = buf[0, pl.ds(c * 16, 16)]
      for t in range(1, _NT):
        v = v + buf[t, pl.ds(c * 16, 16)]
      accl[pl.ds(c * 16, 16)] = v

    def repl_body(i, carry):
      v = accl[pl.ds(i * 16, 16)]
      for j in range(16):
        repl[i * 16 + j] = jnp.full((16,), v[j], jnp.float32)
      return carry

    lax.fori_loop(0, 640 // 16, repl_body, 0)
    pltpu.sync_copy(repl.at[pl.ds(0, _RPT), pl.ds(0, 8)],
                    out_hbm.at[cid].at[pl.ds(r0, _RPT)])

  return sc_degree


def _dinv_block(dp_ref):
  deg = 1.0 + dp_ref[0, :, 0:1] + dp_ref[1, :, 0:1]
  return lax.rsqrt(deg)


def _row_mask(n_valid):
  rid = pl.program_id(0) * _BN + lax.broadcasted_iota(jnp.int32, (_BN, 1), 0)
  return rid < n_valid


def _k2_body(x_ref, w1_ref, dp_ref, y1_ref):
  dinv = _dinv_block(dp_ref)
  xw = jnp.dot(x_ref[...], w1_ref[...], preferred_element_type=jnp.float32)
  y1_ref[...] = jnp.where(_row_mask(_N), xw * dinv, 0.0)


def _k4_body(dp_ref, s1_ref, y1_ref, b1_ref, w2_ref, y2_ref):
  dinv = _dinv_block(dp_ref)
  t = dinv * (s1_ref[0] + s1_ref[1] + y1_ref[...]) + b1_ref[...]
  h = jnp.maximum(t, 0.0)
  y2 = jnp.dot(h, w2_ref[...], preferred_element_type=jnp.float32) * dinv
  y2_ref[...] = jnp.where(_row_mask(_N), y2, 0.0)


def _k6_body(dp_ref, s2_ref, y2_ref, b2_ref, o_ref):
  dinv = _dinv_block(dp_ref)
  o = dinv * (s2_ref[0] + s2_ref[1] + y2_ref[...]) + b2_ref[...]
  m = jnp.max(o, axis=1, keepdims=True)
  lse = m + jnp.log(jnp.sum(jnp.exp(o - m), axis=1, keepdims=True))
  o_ref[...] = o - lse


def kernel(x, edge_index, W1, b1, W2, b2):
  n, d_in = x.shape
  h = W1.shape[1]
  d_out = W2.shape[1]
  e = edge_index.shape[1]

  pad = _NW * _NCH * _CH - e
  dst3 = jnp.concatenate(
      [edge_index[1], jnp.full((pad,), _N, jnp.int32)]).reshape(_NW, _NCH, _CH)

  z1 = jnp.zeros((_NP,), jnp.float32)

  dp = _make_sc_degree()(dst3, z1)
  src3 = jnp.concatenate(
      [edge_index[0], jnp.full((pad,), _N, jnp.int32)]).reshape(_NW, _NCH, _CH)

  grid = (-(-_NP // _BN),)
  y1 = pl.pallas_call(
      _k2_body,
      grid=grid,
      in_specs=[
          pl.BlockSpec((_BN, d_in), lambda i: (i, 0)),
          pl.BlockSpec((d_in, h), lambda i: (0, 0)),
          pl.BlockSpec((2, _BN, 8), lambda i: (0, i, 0)),
      ],
      out_specs=pl.BlockSpec((_BN, h), lambda i: (i, 0)),
      out_shape=jax.ShapeDtypeStruct((_NP, h), jnp.float32),
  )(x, W1, dp)

  s1 = _make_sc_scatter(h)(y1, src3, dst3, jnp.zeros((_NP, h), jnp.float32))

  y2 = pl.pallas_call(
      _k4_body,
      grid=grid,
      in_specs=[
          pl.BlockSpec((2, _BN, 8), lambda i: (0, i, 0)),
          pl.BlockSpec((2, _BN, h), lambda i: (0, i, 0)),
          pl.BlockSpec((_BN, h), lambda i: (i, 0)),
          pl.BlockSpec((1, h), lambda i: (0, 0)),
          pl.BlockSpec((h, d_out), lambda i: (0, 0)),
      ],
      out_specs=pl.BlockSpec((_BN, d_out), lambda i: (i, 0)),
      out_shape=jax.ShapeDtypeStruct((_NP, d_out), jnp.float32),
  )(dp, s1, y1, b1.reshape(1, h), W2)

  s2 = _make_sc_scatter(d_out)(y2, src3, dst3,
                               jnp.zeros((_NP, d_out), jnp.float32))

  out = pl.pallas_call(
      _k6_body,
      grid=grid,
      in_specs=[
          pl.BlockSpec((2, _BN, 8), lambda i: (0, i, 0)),
          pl.BlockSpec((2, _BN, d_out), lambda i: (0, i, 0)),
          pl.BlockSpec((_BN, d_out), lambda i: (i, 0)),
          pl.BlockSpec((1, d_out), lambda i: (0, 0)),
      ],
      out_specs=pl.BlockSpec((_BN, d_out), lambda i: (i, 0)),
      out_shape=jax.ShapeDtypeStruct((n, d_out), jnp.float32),
  )(dp, s2, y2, b2.reshape(1, d_out))

  return out


# speedup vs baseline: 1.0177x; 1.0177x over previous
import functools

import jax
import jax.numpy as jnp
from jax import lax
from jax.experimental import pallas as pl
from jax.experimental.pallas import tpu as pltpu
from jax.experimental.pallas import tpu_sc as plsc

_N = 10000
_NP = 10112
_NT = 16
_NC = 2
_NW = _NT * _NC
_RPT = _NP // _NT
_CH = 512
_NCH = 20
_NBUF = 2
_BN = 2048


def _make_sc_scatter(d: int):
  mesh = plsc.VectorSubcoreMesh(core_axis_name="c", subcore_axis_name="s")

  @functools.partial(
      pl.kernel,
      out_type=jax.ShapeDtypeStruct((_NC, _NP, d), jnp.float32),
      mesh=mesh,
      compiler_params=pltpu.CompilerParams(use_tc_tiling_on_sc=False),
      scratch_types=[
          pltpu.VMEM((_NCH, _CH), jnp.int32),
          pltpu.VMEM((_NCH, _CH), jnp.int32),
          pltpu.VMEM((_NBUF, _CH, d), jnp.float32),
          pltpu.VMEM_SHARED((_NP, d), jnp.float32),
          pltpu.VMEM_SHARED((_NP, d), jnp.float32),
          [pltpu.SemaphoreType.DMA] * _NBUF,
          [pltpu.SemaphoreType.DMA] * _NBUF,
      ],
  )
  def sc_scatter(table_hbm, src_hbm, dst_hbm, zeros_hbm, out_hbm,
                 src_v, dst_v, rows_v, acc, table_spm, gsems, ssems):
    cid = lax.axis_index("c")
    sid = lax.axis_index("s")
    wid = sid * _NC + cid
    r0 = sid * _RPT
    pltpu.sync_copy(zeros_hbm.at[pl.ds(r0, _RPT)], acc.at[pl.ds(r0, _RPT)])
    pltpu.sync_copy(table_hbm.at[pl.ds(r0, _RPT)], table_spm.at[pl.ds(r0, _RPT)])
    pltpu.sync_copy(src_hbm.at[wid], src_v)
    pltpu.sync_copy(dst_hbm.at[wid], dst_v)
    plsc.subcore_barrier()

    gh = [None] * _NBUF
    sh = [None] * _NBUF

    def issue_gather(c):
      b = c % _NBUF
      if sh[b] is not None:
        sh[b].wait()
        sh[b] = None
      gh[b] = pltpu.async_copy(
          table_spm.at[src_v.at[c]], rows_v.at[b], gsems[b])

    for c in range(min(_NBUF, _NCH)):
      issue_gather(c)
    for c in range(_NCH):
      b = c % _NBUF
      gh[b].wait()
      sh[b] = pltpu.async_copy(rows_v.at[b], acc.at[dst_v.at[c]], ssems[b],
                               add=True)
      if c + _NBUF < _NCH:
        issue_gather(c + _NBUF)
    for b in range(_NBUF):
      if sh[b] is not None:
        sh[b].wait()
    plsc.subcore_barrier()
    pltpu.sync_copy(acc.at[pl.ds(r0, _RPT)],
                    out_hbm.at[cid].at[pl.ds(r0, _RPT)])

  return sc_scatter


def _make_sc_degree():
  mesh = plsc.VectorSubcoreMesh(core_axis_name="c", subcore_axis_name="s")

  @functools.partial(
      pl.kernel,
      out_type=jax.ShapeDtypeStruct((_NC, _NP, 16), jnp.float32),
      mesh=mesh,
      compiler_params=pltpu.CompilerParams(use_tc_tiling_on_sc=False,
                                           needs_layout_passes=False),
      scratch_types=[
          pltpu.VMEM((_NCH, _CH), jnp.int32),
          pltpu.VMEM((_NP,), jnp.float32),
          pltpu.VMEM((_NT, 640), jnp.float32),
          pltpu.VMEM((640,), jnp.float32),
          pltpu.VMEM((640, 16), jnp.float32),
          pltpu.VMEM_SHARED((_NT, _NP), jnp.float32),
      ],
  )
  def sc_degree(dst_hbm, zeros_hbm, out_hbm, dst_v, degl, buf, accl, repl,
                stage):
    cid = lax.axis_index("c")
    sid = lax.axis_index("s")
    wid = sid * _NC + cid
    r0 = sid * _RPT
    pltpu.sync_copy(dst_hbm.at[wid], dst_v)
    pltpu.sync_copy(zeros_hbm, degl)
    ones = jnp.ones((16,), jnp.float32)

    def body(r, carry):
      for k in range(_CH // 16):
        idx = dst_v[r, pl.ds(k * 16, 16)]
        plsc.addupdate_scatter(degl, [idx], ones)
      return carry

    lax.fori_loop(0, _NCH, body, 0)
    pltpu.sync_copy(degl, stage.at[sid])
    plsc.subcore_barrier()
    for t in range(_NT):
      pltpu.sync_copy(stage.at[t].at[pl.ds(r0, _RPT)],
                      buf.at[t].at[pl.ds(0, _RPT)])
    for c in range(640 // 16):
      v = buf[0, pl.ds(c * 16, 16)]
      for t in range(1, _NT):
        v = v + buf[t, pl.ds(c * 16, 16)]
      accl[pl.ds(c * 16, 16)] = v

    def repl_body(i, carry):
      v = accl[pl.ds(i * 16, 16)]
      for j in range(16):
        repl[i * 16 + j] = jnp.full((16,), v[j], jnp.float32)
      return carry

    lax.fori_loop(0, 640 // 16, repl_body, 0)
    pltpu.sync_copy(repl.at[pl.ds(0, _RPT)],
                    out_hbm.at[cid].at[pl.ds(r0, _RPT)])

  return sc_degree


def _dinv_block(dp_ref):
  deg = 1.0 + dp_ref[0, :, 0:1] + dp_ref[1, :, 0:1]
  return lax.rsqrt(deg)


def _row_mask(n_valid):
  rid = pl.program_id(0) * _BN + lax.broadcasted_iota(jnp.int32, (_BN, 1), 0)
  return rid < n_valid


def _k2_body(x_ref, w1_ref, dp_ref, y1_ref):
  dinv = _dinv_block(dp_ref)
  xw = jnp.dot(x_ref[...], w1_ref[...], preferred_element_type=jnp.float32)
  y1_ref[...] = jnp.where(_row_mask(_N), xw * dinv, 0.0)


def _k4_body(dp_ref, s1_ref, y1_ref, b1_ref, w2_ref, y2_ref):
  dinv = _dinv_block(dp_ref)
  t = dinv * (s1_ref[0] + s1_ref[1] + y1_ref[...]) + b1_ref[...]
  h = jnp.maximum(t, 0.0)
  y2 = jnp.dot(h, w2_ref[...], preferred_element_type=jnp.float32) * dinv
  y2_ref[...] = jnp.where(_row_mask(_N), y2, 0.0)


def _k6_body(dp_ref, s2_ref, y2_ref, b2_ref, o_ref):
  dinv = _dinv_block(dp_ref)
  o = dinv * (s2_ref[0] + s2_ref[1] + y2_ref[...]) + b2_ref[...]
  m = jnp.max(o, axis=1, keepdims=True)
  lse = m + jnp.log(jnp.sum(jnp.exp(o - m), axis=1, keepdims=True))
  o_ref[...] = o - lse


def kernel(x, edge_index, W1, b1, W2, b2):
  n, d_in = x.shape
  h = W1.shape[1]
  d_out = W2.shape[1]
  e = edge_index.shape[1]

  pad = _NW * _NCH * _CH - e
  dst3 = jnp.concatenate(
      [edge_index[1], jnp.full((pad,), _N, jnp.int32)]).reshape(_NW, _NCH, _CH)

  z1 = jnp.zeros((_NP,), jnp.float32)

  dp = _make_sc_degree()(dst3, z1)
  src3 = jnp.concatenate(
      [edge_index[0], jnp.full((pad,), _N, jnp.int32)]).reshape(_NW, _NCH, _CH)

  grid = (-(-_NP // _BN),)
  y1 = pl.pallas_call(
      _k2_body,
      grid=grid,
      in_specs=[
          pl.BlockSpec((_BN, d_in), lambda i: (i, 0)),
          pl.BlockSpec((d_in, h), lambda i: (0, 0)),
          pl.BlockSpec((2, _BN, 16), lambda i: (0, i, 0)),
      ],
      out_specs=pl.BlockSpec((_BN, h), lambda i: (i, 0)),
      out_shape=jax.ShapeDtypeStruct((_NP, h), jnp.float32),
  )(x, W1, dp)

  s1 = _make_sc_scatter(h)(y1, src3, dst3, jnp.zeros((_NP, h), jnp.float32))

  y2 = pl.pallas_call(
      _k4_body,
      grid=grid,
      in_specs=[
          pl.BlockSpec((2, _BN, 16), lambda i: (0, i, 0)),
          pl.BlockSpec((2, _BN, h), lambda i: (0, i, 0)),
          pl.BlockSpec((_BN, h), lambda i: (i, 0)),
          pl.BlockSpec((1, h), lambda i: (0, 0)),
          pl.BlockSpec((h, d_out), lambda i: (0, 0)),
      ],
      out_specs=pl.BlockSpec((_BN, d_out), lambda i: (i, 0)),
      out_shape=jax.ShapeDtypeStruct((_NP, d_out), jnp.float32),
  )(dp, s1, y1, b1.reshape(1, h), W2)

  s2 = _make_sc_scatter(d_out)(y2, src3, dst3,
                               jnp.zeros((_NP, d_out), jnp.float32))

  out = pl.pallas_call(
      _k6_body,
      grid=grid,
      in_specs=[
          pl.BlockSpec((2, _BN, 16), lambda i: (0, i, 0)),
          pl.BlockSpec((2, _BN, d_out), lambda i: (0, i, 0)),
          pl.BlockSpec((_BN, d_out), lambda i: (i, 0)),
          pl.BlockSpec((1, d_out), lambda i: (0, 0)),
      ],
      out_specs=pl.BlockSpec((_BN, d_out), lambda i: (i, 0)),
      out_shape=jax.ShapeDtypeStruct((n, d_out), jnp.float32),
  )(dp, s2, y2, b2.reshape(1, d_out))

  return out
